# baseline (speedup 1.0000x reference)
import functools

import jax
import jax.numpy as jnp
from jax import lax
from jax.experimental import pallas as pl
from jax.experimental.pallas import tpu as pltpu
from jax.experimental.pallas import tpu_sc as plsc

NC, NS = 2, 16
NW = NC * NS
D = 64
V = 1000000
VFULL = (V // 128) * 128
NBLK = VFULL // 128
BLK_PER_W = -(-NBLK // NW)

_mesh = lambda: plsc.VectorSubcoreMesh(core_axis_name="c", subcore_axis_name="s")


def _iota16():
    return lax.iota(jnp.int32, 16)


def _make_transpose():
    @functools.partial(
        pl.kernel,
        mesh=_mesh(),
        out_type=jax.ShapeDtypeStruct((V // 2, 128), jnp.float32),
        compiler_params=pltpu.CompilerParams(needs_layout_passes=False),
        scratch_types=[
            pltpu.VMEM((D, 128), jnp.float32),
            pltpu.VMEM((D, 128), jnp.float32),
            pltpu.VMEM((D, D), jnp.float32),
        ],
    )
    def transpose_k(tabT_hbm, tail_hbm, scr_hbm, inb, outb, tailb):
        wid = lax.axis_index("s") * NC + lax.axis_index("c")
        iot = _iota16()

        def do_block(blk):
            pltpu.sync_copy(tabT_hbm.at[:, pl.ds(blk * 128, 128)], inb)

            def row(ql, _):
                for k in range(8):
                    rows = iot + (16 * (k % 4))
                    cols = jnp.zeros((16,), jnp.int32) + (2 * ql + (k // 4))
                    v = plsc.load_gather(inb, [rows, cols])
                    outb[ql, 16 * k : 16 * k + 16] = v
                return 0

            lax.fori_loop(0, D, row, 0)
            pltpu.sync_copy(outb, scr_hbm.at[pl.ds(blk * D, D), :])

        def loop_i(i, _):
            blk = wid + NW * i

            @pl.when(blk < NBLK)
            def _():
                do_block(blk)

            return 0

        lax.fori_loop(0, BLK_PER_W, loop_i, 0)

        @pl.when(wid == NW - 1)
        def _():
            pltpu.sync_copy(tail_hbm, tailb)

            def trow(q, _):
                for k in range(8):
                    rows = jnp.zeros((16,), jnp.int32) + (2 * q + (k // 4))
                    cols = iot + (16 * (k % 4))
                    v = plsc.load_gather(tailb, [rows, cols])
                    outb[q, 16 * k : 16 * k + 16] = v
                return 0

            lax.fori_loop(0, (V - VFULL) // 2, trow, 0)
            pltpu.sync_copy(
                outb.at[pl.ds(0, (V - VFULL) // 2), :],
                scr_hbm.at[pl.ds(VFULL // 2, (V - VFULL) // 2), :],
            )

    return transpose_k


def _make_gather(batch: int, seq: int):
    b_per_w = batch // NW

    @functools.partial(
        pl.kernel,
        mesh=_mesh(),
        out_type=jax.ShapeDtypeStruct((seq, D, batch), jnp.float32),
        compiler_params=pltpu.CompilerParams(needs_layout_passes=False),
        scratch_types=[
            pltpu.VMEM((8, b_per_w), jnp.int32),
            pltpu.VMEM((8, b_per_w), jnp.int32),
            pltpu.VMEM((8, b_per_w), jnp.int32),
            pltpu.VMEM((b_per_w, 128), jnp.float32),
            pltpu.VMEM((D, b_per_w), jnp.float32),
            pltpu.SemaphoreType.DMA,
        ],
    )
    def gather_k(idsT_hbm, scr_hbm, out_hbm, ids_v, pidx_v, off_v, buf, blkb, gsem):
        wid = lax.axis_index("s") * NC + lax.axis_index("c")
        b0 = wid * b_per_w
        iot = _iota16()

        def s8_loop(s8i, _):
            s8 = s8i * 8
            pltpu.sync_copy(idsT_hbm.at[pl.ds(s8, 8), pl.ds(b0, b_per_w)], ids_v)

            def prep(r, _):
                for k in range(b_per_w // 16):
                    v = ids_v[r, 16 * k : 16 * k + 16]
                    pidx_v[r, 16 * k : 16 * k + 16] = v >> 1
                    off_v[r, 16 * k : 16 * k + 16] = (v & 1) << 6
                return 0

            lax.fori_loop(0, 8, prep, 0)

            def srow(r, _):
                pltpu.async_copy(scr_hbm.at[pidx_v.at[r]], buf, gsem).wait()

                def fcol(f, _):
                    for g in range(b_per_w // 16):
                        rows = iot + 16 * g
                        cols = off_v[r, 16 * g : 16 * g + 16] + f
                        v = plsc.load_gather(buf, [rows, cols])
                        blkb[f, 16 * g : 16 * g + 16] = v
                    return 0

                lax.fori_loop(0, D, fcol, 0)
                pltpu.sync_copy(blkb, out_hbm.at[s8 + r, :, pl.ds(b0, b_per_w)])
                return 0

            lax.fori_loop(0, 8, srow, 0)
            return 0

        lax.fori_loop(0, seq // 8, s8_loop, 0)

    return gather_k


def kernel(input_ids, table):
    batch, seq = input_ids.shape
    ids32T = input_ids.astype(jnp.int32).T
    tabT = table.T
    tail = table[VFULL:, :]
    scr = _make_transpose()(tabT, tail)
    outT = _make_gather(batch, seq)(ids32T, scr)
    return outT.transpose(2, 0, 1)


# speedup vs baseline: 2.6309x; 2.6309x over previous
import functools

import jax
import jax.numpy as jnp
from jax import lax
from jax.experimental import pallas as pl
from jax.experimental.pallas import tpu as pltpu
from jax.experimental.pallas import tpu_sc as plsc

NC, NS = 2, 16
NW = NC * NS
D = 64
V = 1000000
VFULL = (V // 128) * 128
NBLK = VFULL // 128
NI = 246

_mesh = lambda: plsc.VectorSubcoreMesh(core_axis_name="c", subcore_axis_name="s")


def _iota16():
    return lax.iota(jnp.int32, 16)


def _make_transpose():
    @functools.partial(
        pl.kernel,
        mesh=_mesh(),
        out_type=jax.ShapeDtypeStruct((V // 2, 128), jnp.float32),
        compiler_params=pltpu.CompilerParams(needs_layout_passes=False),
        scratch_types=[
            pltpu.VMEM((D, 128), jnp.float32),
            pltpu.VMEM((D, 128), jnp.float32),
            pltpu.VMEM((D, 128), jnp.float32),
            pltpu.VMEM((D, 128), jnp.float32),
            pltpu.VMEM((D, D), jnp.float32),
            pltpu.SemaphoreType.DMA,
            pltpu.SemaphoreType.DMA,
        ],
    )
    def transpose_k(tabT_hbm, tail_hbm, scr_hbm, in0, in1, ou0, ou1, tailb, isem, osem):
        wid = lax.axis_index("s") * NC + lax.axis_index("c")
        iot = _iota16()
        rows_g = [(iot >> 1) + 8 * g for g in range(8)]
        colbase = (iot & 1) << 6
        ins = (in0, in1)
        ous = (ou0, ou1)

        def blk_of(i):
            return wid + NW * i

        def transpose_block(inb, outb):
            @plsc.parallel_loop(0, D, unroll=4)
            def _(f):
                cols_f = colbase + f
                for g in range(8):
                    v = inb[f, 16 * g : 16 * g + 16]
                    plsc.store_scatter(outb, [rows_g[g], cols_f], v)

        @pl.when(blk_of(0) < NBLK)
        def _():
            pltpu.async_copy(tabT_hbm.at[:, pl.ds(blk_of(0) * 128, 128)], in0, isem)

        def pair_body(p, _):
            for b in range(2):
                i = 2 * p + b

                @pl.when(blk_of(i) < NBLK)
                def _():
                    pltpu.make_async_copy(
                        tabT_hbm.at[:, pl.ds(0, 128)], ins[b], isem
                    ).wait()

                    @pl.when(blk_of(i + 1) < NBLK)
                    def _():
                        pltpu.async_copy(
                            tabT_hbm.at[:, pl.ds(blk_of(i + 1) * 128, 128)],
                            ins[1 - b],
                            isem,
                        )

                    @pl.when(i >= 2)
                    def _():
                        pltpu.make_async_copy(
                            ous[b], scr_hbm.at[pl.ds(0, D), :], osem
                        ).wait()

                    transpose_block(ins[b], ous[b])
                    pltpu.async_copy(
                        ous[b], scr_hbm.at[pl.ds(blk_of(i) * D, D), :], osem
                    )

            return 0

        lax.fori_loop(0, NI // 2, pair_body, 0)
        for b in range(2):
            pltpu.make_async_copy(ous[b], scr_hbm.at[pl.ds(0, D), :], osem).wait()

        @pl.when(wid == NW - 1)
        def _():
            pltpu.sync_copy(tail_hbm, tailb)
            ntr = (V - VFULL) // 2

            def trow(q, _):
                for k in range(8):
                    rows = jnp.zeros((16,), jnp.int32) + (2 * q + (k // 4))
                    cols = iot + (16 * (k % 4))
                    v = plsc.load_gather(tailb, [rows, cols])
                    ou0[q, 16 * k : 16 * k + 16] = v
                return 0

            lax.fori_loop(0, ntr, trow, 0)
            pltpu.sync_copy(
                ou0.at[pl.ds(0, ntr), :],
                scr_hbm.at[pl.ds(VFULL // 2, ntr), :],
            )

    return transpose_k


def _make_gather(batch: int, seq: int):
    bw = batch // NW

    @functools.partial(
        pl.kernel,
        mesh=_mesh(),
        out_type=jax.ShapeDtypeStruct((seq, D, batch), jnp.float32),
        compiler_params=pltpu.CompilerParams(needs_layout_passes=False),
        scratch_types=[
            pltpu.VMEM((8, bw), jnp.int32),
            pltpu.VMEM((8, bw), jnp.int32),
            pltpu.VMEM((8, bw), jnp.int32),
            pltpu.VMEM((bw, 128), jnp.float32),
            pltpu.VMEM((bw, 128), jnp.float32),
            pltpu.VMEM((D, bw), jnp.float32),
            pltpu.VMEM((D, bw), jnp.float32),
            pltpu.SemaphoreType.DMA,
            pltpu.SemaphoreType.DMA,
        ],
    )
    def gather_k(idsT_hbm, scr_hbm, out_hbm, ids_v, pidx_v, off_v, bf0, bf1, bl0, bl1, gsem, wsem):
        wid = lax.axis_index("s") * NC + lax.axis_index("c")
        b0 = wid * bw
        iot = _iota16()
        rows_g = [iot + 16 * g for g in range(8)]
        bufs = (bf0, bf1)
        blks = (bl0, bl1)
        ng = bw // 16

        def s8_loop(s8i, _):
            s8 = s8i * 8
            pltpu.sync_copy(idsT_hbm.at[pl.ds(s8, 8), pl.ds(b0, bw)], ids_v)

            @plsc.parallel_loop(0, 8, unroll=2)
            def _(r):
                for k in range(ng):
                    v = ids_v[r, 16 * k : 16 * k + 16]
                    pidx_v[r, 16 * k : 16 * k + 16] = v >> 1
                    off_v[r, 16 * k : 16 * k + 16] = (v & 1) << 6

            def fire(r):
                return pltpu.async_copy(
                    scr_hbm.at[pidx_v.at[r]], bufs[r & 1], gsem
                )

            def drain_w(b):
                pltpu.make_async_copy(
                    blks[b], out_hbm.at[0, :, pl.ds(0, bw)], wsem
                ).wait()

            descs = {0: fire(0)}
            for r in range(8):
                b = r & 1
                if r < 7:
                    descs[r + 1] = fire(r + 1)
                if r >= 2:
                    drain_w(b)
                else:
                    @pl.when(s8i > 0)
                    def _():
                        drain_w(b)

                descs[r].wait()
                buf = bufs[b]
                blkb = blks[b]
                off_list = [off_v[r, 16 * g : 16 * g + 16] for g in range(ng)]

                @plsc.parallel_loop(0, D, unroll=4)
                def _(f):
                    for g in range(ng):
                        v = plsc.load_gather(buf, [rows_g[g], off_list[g] + f])
                        blkb[f, 16 * g : 16 * g + 16] = v

                pltpu.async_copy(blkb, out_hbm.at[s8 + r, :, pl.ds(b0, bw)], wsem)
            return 0

        lax.fori_loop(0, seq // 8, s8_loop, 0)
        for b in range(2):
            pltpu.make_async_copy(blks[b], out_hbm.at[0, :, pl.ds(0, bw)], wsem).wait()

    return gather_k


def kernel(input_ids, table):
    batch, seq = input_ids.shape
    ids32T = input_ids.astype(jnp.int32).T
    tabT = table.T
    tail = table[VFULL:, :]
    scr = _make_transpose()(tabT, tail)
    outT = _make_gather(batch, seq)(ids32T, scr)
    return outT.transpose(2, 0, 1)
